# baseline (speedup 1.0000x reference)
import jax
import jax.numpy as jnp
from jax import lax
from jax.experimental import pallas as pl
from jax.experimental.pallas import tpu as pltpu

N_EXPERTS = 64
TOP_K = 2
BLOCK_N = 512


def _router_kernel(x_ref, w_ref, b_ref, gates_ref, idx_ref):
    logits = lax.dot_general(
        x_ref[...],
        w_ref[...],
        dimension_numbers=(((1,), (1,)), ((), ())),
        preferred_element_type=jnp.float32,
    ) + b_ref[...]

    lt = logits.T
    eT = lax.broadcasted_iota(jnp.int32, lt.shape, 0)

    m1 = jnp.max(lt, axis=0, keepdims=True)
    i1 = jnp.min(jnp.where(lt == m1, eT, N_EXPERTS), axis=0, keepdims=True)

    masked = jnp.where(eT == i1, -jnp.inf, lt)
    m2 = jnp.max(masked, axis=0, keepdims=True)
    i2 = jnp.min(jnp.where(masked == m2, eT, N_EXPERTS), axis=0, keepdims=True)

    e2 = jnp.exp(m2 - m1)
    denom = 1.0 + e2
    p1 = 1.0 / denom
    p2 = e2 / denom

    gates_ref[...] = jnp.where(eT == i1, p1, 0.0) + jnp.where(eT == i2, p2, 0.0)
    idx_ref[...] = jnp.concatenate([i1, i2], axis=0)


@jax.jit
def kernel(x, W, b):
    n, d = x.shape
    grid = (n // BLOCK_N,)
    gates_t, idx_t = pl.pallas_call(
        _router_kernel,
        grid=grid,
        in_specs=[
            pl.BlockSpec((BLOCK_N, d), lambda i: (i, 0)),
            pl.BlockSpec((N_EXPERTS, d), lambda i: (0, 0)),
            pl.BlockSpec((1, N_EXPERTS), lambda i: (0, 0)),
        ],
        out_specs=[
            pl.BlockSpec((N_EXPERTS, BLOCK_N), lambda i: (0, i)),
            pl.BlockSpec((TOP_K, BLOCK_N), lambda i: (0, i)),
        ],
        out_shape=[
            jax.ShapeDtypeStruct((N_EXPERTS, n), jnp.float32),
            jax.ShapeDtypeStruct((TOP_K, n), jnp.int32),
        ],
        compiler_params=pltpu.CompilerParams(
            dimension_semantics=("parallel",),
            vmem_limit_bytes=50 * 1024 * 1024,
        ),
    )(x, W.T, b.reshape(1, N_EXPERTS))
    return (gates_t.T, idx_t.T)


# speedup vs baseline: 1.1962x; 1.1962x over previous
import jax
import jax.numpy as jnp
from jax import lax
from jax.experimental import pallas as pl
from jax.experimental.pallas import tpu as pltpu

N_EXPERTS = 64
TOP_K = 2
BLOCK_N = 1024


def _router_kernel(x_ref, w_ref, b_ref, gates_ref, idx_ref):
    logits = lax.dot_general(
        x_ref[...],
        w_ref[...],
        dimension_numbers=(((1,), (1,)), ((), ())),
        preferred_element_type=jnp.float32,
    ) + b_ref[...]

    lt = logits.T
    eT = lax.broadcasted_iota(jnp.int32, lt.shape, 0)

    m1 = jnp.max(lt, axis=0, keepdims=True)
    i1 = jnp.min(jnp.where(lt == m1, eT, N_EXPERTS), axis=0, keepdims=True)

    masked = jnp.where(eT == i1, -jnp.inf, lt)
    m2 = jnp.max(masked, axis=0, keepdims=True)
    i2 = jnp.min(jnp.where(masked == m2, eT, N_EXPERTS), axis=0, keepdims=True)

    e2 = jnp.exp(m2 - m1)
    denom = 1.0 + e2
    p1 = 1.0 / denom
    p2 = e2 / denom

    gates_ref[...] = jnp.where(eT == i1, p1, 0.0) + jnp.where(eT == i2, p2, 0.0)
    idx_ref[...] = jnp.concatenate([i1, i2], axis=0)


@jax.jit
def kernel(x, W, b):
    n, d = x.shape
    grid = (n // BLOCK_N,)
    gates_t, idx_t = pl.pallas_call(
        _router_kernel,
        grid=grid,
        in_specs=[
            pl.BlockSpec((BLOCK_N, d), lambda i: (i, 0)),
            pl.BlockSpec((N_EXPERTS, d), lambda i: (0, 0)),
            pl.BlockSpec((1, N_EXPERTS), lambda i: (0, 0)),
        ],
        out_specs=[
            pl.BlockSpec((N_EXPERTS, BLOCK_N), lambda i: (0, i)),
            pl.BlockSpec((TOP_K, BLOCK_N), lambda i: (0, i)),
        ],
        out_shape=[
            jax.ShapeDtypeStruct((N_EXPERTS, n), jnp.float32),
            jax.ShapeDtypeStruct((TOP_K, n), jnp.int32),
        ],
        compiler_params=pltpu.CompilerParams(
            dimension_semantics=("parallel",),
            vmem_limit_bytes=50 * 1024 * 1024,
        ),
    )(x, W.T, b.reshape(1, N_EXPERTS))
    return (gates_t.T, idx_t.T)
